# baseline (speedup 1.0000x reference)
import functools

import jax
import jax.numpy as jnp
from jax import lax
from jax.experimental import pallas as pl
from jax.experimental.pallas import tpu as pltpu
from jax.experimental.pallas import tpu_sc as plsc

N_PER_GRAPH = 16384
NGRAPH = 4
N = NGRAPH * N_PER_GRAPH
E = 524288
D0 = 128
D1 = 256

NC = 2
NS = 16
NW = NC * NS
F = 64
FC = 128
SLABR = 7168
NSLAB = (N + SLABR - 1) // SLABR
ZR = 64


_ERF_P = 0.3275911
_ERF_A1 = 0.254829592
_ERF_A2 = -0.284496736
_ERF_A3 = 1.421413741
_ERF_A4 = -1.453152027
_ERF_A5 = 1.061405429


def _gelu(x):
    z = jnp.abs(x) * 0.7071067811865476
    t = 1.0 / (1.0 + _ERF_P * z)
    poly = t * (_ERF_A1 + t * (_ERF_A2 + t * (_ERF_A3 + t * (_ERF_A4 + t * _ERF_A5))))
    erf = jnp.sign(x) * (1.0 - poly * jnp.exp(-z * z))
    return 0.5 * x * (1.0 + erf)


def _tc_scale(msg, w):
    BLK = 8192

    def body(m_ref, w_ref, o_ref):
        o_ref[...] = m_ref[...] * w_ref[...]

    return pl.pallas_call(
        body,
        grid=(E // BLK,),
        in_specs=[
            pl.BlockSpec((BLK, D0), lambda i: (i, 0)),
            pl.BlockSpec((BLK, 1), lambda i: (i, 0)),
        ],
        out_specs=pl.BlockSpec((BLK, D0), lambda i: (i, 0)),
        out_shape=jax.ShapeDtypeStruct((E, D0), jnp.float32),
    )(msg, w.reshape(E, 1))


def _tc_layer1(h, agg, W, b):
    BLK = 2048

    def body(h_ref, a_ref, w_ref, b_ref, oa_ref, ob_ref):
        x = h_ref[...] + a_ref[...]
        y = _gelu(jnp.dot(x, w_ref[...], preferred_element_type=jnp.float32)
                  + b_ref[...])
        oa_ref[...] = y[:, :D0]
        ob_ref[...] = y[:, D0:]

    return pl.pallas_call(
        body,
        grid=(N // BLK,),
        in_specs=[
            pl.BlockSpec((BLK, D0), lambda i: (i, 0)),
            pl.BlockSpec((BLK, D0), lambda i: (i, 0)),
            pl.BlockSpec((D0, D1), lambda i: (0, 0)),
            pl.BlockSpec((1, D1), lambda i: (0, 0)),
        ],
        out_specs=[
            pl.BlockSpec((BLK, D0), lambda i: (i, 0)),
            pl.BlockSpec((BLK, D0), lambda i: (i, 0)),
        ],
        out_shape=[
            jax.ShapeDtypeStruct((N, D0), jnp.float32),
            jax.ShapeDtypeStruct((N, D0), jnp.float32),
        ],
    )(h, agg, W, b.reshape(1, D1))


def _tc_layer23(u1a, u1b, agg2a, agg2b, W2, b2, W3):
    BLK = 2048

    def body(ha_ref, hb_ref, aa_ref, ab_ref, w2a_ref, w2b_ref, b2_ref,
             w3_ref, o_ref):
        xa = ha_ref[...] + aa_ref[...]
        xb = hb_ref[...] + ab_ref[...]
        y = jnp.dot(xa, w2a_ref[...], preferred_element_type=jnp.float32)
        y = y + jnp.dot(xb, w2b_ref[...], preferred_element_type=jnp.float32)
        y = _gelu(y + b2_ref[...])
        o_ref[...] = jnp.dot(y, w3_ref[...], preferred_element_type=jnp.float32)

    return pl.pallas_call(
        body,
        grid=(N // BLK,),
        in_specs=[
            pl.BlockSpec((BLK, D0), lambda i: (i, 0)),
            pl.BlockSpec((BLK, D0), lambda i: (i, 0)),
            pl.BlockSpec((BLK, D0), lambda i: (i, 0)),
            pl.BlockSpec((BLK, D0), lambda i: (i, 0)),
            pl.BlockSpec((D0, D1), lambda i: (0, 0)),
            pl.BlockSpec((D0, D1), lambda i: (0, 0)),
            pl.BlockSpec((1, D1), lambda i: (0, 0)),
            pl.BlockSpec((D1, D0), lambda i: (0, 0)),
        ],
        out_specs=pl.BlockSpec((BLK, D0), lambda i: (i, 0)),
        out_shape=jax.ShapeDtypeStruct((N, D0), jnp.float32),
    )(u1a, u1b, agg2a, agg2b, W2[:D0], W2[D0:], b2.reshape(1, D1), W3)


def _tc_layer3_pool(m, agg, b3):
    BLK = 2048
    NBLK = N // BLK

    def body(m_ref, a_ref, b_ref, o_ref):
        y = _gelu(m_ref[...] + a_ref[...] + b_ref[...])
        o_ref[...] = jnp.sum(y, axis=0).reshape(1, 1, D0)

    return pl.pallas_call(
        body,
        grid=(NBLK,),
        in_specs=[
            pl.BlockSpec((BLK, D0), lambda i: (i, 0)),
            pl.BlockSpec((BLK, D0), lambda i: (i, 0)),
            pl.BlockSpec((1, D0), lambda i: (0, 0)),
        ],
        out_specs=pl.BlockSpec((1, 1, D0), lambda i: (i, 0, 0)),
        out_shape=jax.ShapeDtypeStruct((NBLK, 1, D0), jnp.float32),
    )(m, agg, b3.reshape(1, D0))


def _tc_mix(psum, bert_feat, Wb, bb, gamma, beta):
    def body(ps_ref, bf_ref, wb_ref, bb_ref, g_ref, be_ref, o_ref):
        pool = jnp.sum(ps_ref[...], axis=1) * (1.0 / N_PER_GRAPH)
        enc = jnp.dot(bf_ref[...], wb_ref[...],
                      preferred_element_type=jnp.float32) + bb_ref[...]
        mu = jnp.mean(enc, axis=-1, keepdims=True)
        var = jnp.mean((enc - mu) ** 2, axis=-1, keepdims=True)
        enc = (enc - mu) / jnp.sqrt(var + 1e-5) * g_ref[...] + be_ref[...]
        o_ref[...] = (pool + enc) * 0.5

    nblk = psum.shape[1]
    return pl.pallas_call(
        body,
        in_specs=[
            pl.BlockSpec((NGRAPH, nblk, D0), lambda: (0, 0, 0)),
            pl.BlockSpec((NGRAPH, 1536), lambda: (0, 0)),
            pl.BlockSpec((1536, D0), lambda: (0, 0)),
            pl.BlockSpec((1, D0), lambda: (0, 0)),
            pl.BlockSpec((1, D0), lambda: (0, 0)),
            pl.BlockSpec((1, D0), lambda: (0, 0)),
        ],
        out_specs=pl.BlockSpec((NGRAPH, D0), lambda: (0, 0)),
        out_shape=jax.ShapeDtypeStruct((NGRAPH, D0), jnp.float32),
    )(psum, bert_feat, Wb, bb.reshape(1, D0), gamma.reshape(1, D0),
      beta.reshape(1, D0))


def _make_gather(NROWS):
    rows_per_w = NROWS // NW
    n_ch = rows_per_w // F
    mesh = plsc.VectorSubcoreMesh(core_axis_name="c", subcore_axis_name="s")

    @functools.partial(
        pl.kernel,
        mesh=mesh,
        out_type=jax.ShapeDtypeStruct((NROWS, D0), jnp.float32),
        scratch_types=[
            pltpu.VMEM((F,), jnp.int32),
            pltpu.VMEM((F, D0), jnp.float32),
            pltpu.VMEM((F,), jnp.int32),
            pltpu.VMEM((F, D0), jnp.float32),
            pltpu.SemaphoreType.DMA,
            pltpu.SemaphoreType.DMA,
        ],
    )
    def k(table_hbm, idx_hbm, out_hbm, ia, ra, ib, rb, sa, sb):
        wid = lax.axis_index("s") * NC + lax.axis_index("c")
        base = wid * rows_per_w

        def body(i, _):
            offa = base + (2 * i) * F
            offb = offa + F
            pltpu.sync_copy(idx_hbm.at[pl.ds(offa, F)], ia)
            cpa = pltpu.async_copy(table_hbm.at[ia], ra, sa)
            pltpu.sync_copy(idx_hbm.at[pl.ds(offb, F)], ib)
            cpb = pltpu.async_copy(table_hbm.at[ib], rb, sb)
            cpa.wait()
            pltpu.sync_copy(ra, out_hbm.at[pl.ds(offa, F)])
            cpb.wait()
            pltpu.sync_copy(rb, out_hbm.at[pl.ds(offb, F)])
            return 0

        lax.fori_loop(0, n_ch // 2, body, 0)

    return k


_gather_nodes = _make_gather(N)
_gather_edges = _make_gather(E)


def _sc_scatter(msgw, dst):
    ER = E // NS
    NSPC = NSLAB // NC
    NB = ER // FC
    mesh = plsc.VectorSubcoreMesh(core_axis_name="c", subcore_axis_name="s")

    @functools.partial(
        pl.kernel,
        mesh=mesh,
        out_type=jax.ShapeDtypeStruct((NSLAB * SLABR, D0), jnp.float32),
        scratch_types=[
            pltpu.VMEM((FC,), jnp.int32),
            pltpu.VMEM((FC,), jnp.int32),
            pltpu.VMEM((FC, D0), jnp.float32),
            pltpu.VMEM((ZR, D0), jnp.float32),
            pltpu.VMEM_SHARED((SLABR + 8, D0), jnp.float32),
            pltpu.SemaphoreType.DMA,
        ],
    )
    def k(msg_hbm, dst_hbm, out_hbm, dstblk, dx, mb, zbuf, slab, sem):
        c = lax.axis_index("c")
        s = lax.axis_index("s")
        zero16 = jnp.zeros((16,), jnp.float32)
        ebase = s * ER

        def zrow(r, _):
            for ccs in range(D0 // 16):
                zbuf[r, pl.ds(ccs * 16, 16)] = zero16
            return 0

        lax.fori_loop(0, ZR, zrow, 0)
        myz0 = s * (SLABR // NS)

        for g in range(NSLAB):
            lo = g * SLABR

            @pl.when(c == (g % NC))
            def _():
                for zi in range(SLABR // NS // ZR):
                    pltpu.sync_copy(zbuf, slab.at[pl.ds(myz0 + zi * ZR, ZR)])
                plsc.subcore_barrier()

                def chunk(f, _):
                    gbase = ebase + f * FC
                    cp = pltpu.async_copy(msg_hbm.at[pl.ds(gbase, FC)], mb,
                                          sem)
                    pltpu.sync_copy(dst_hbm.at[pl.ds(gbase, FC)], dstblk)
                    for q in range(FC // 16):
                        dl = dstblk[pl.ds(q * 16, 16)] - lo
                        ok = (dl >= 0) & (dl < SLABR)
                        dx[pl.ds(q * 16, 16)] = jnp.where(ok, dl, SLABR)
                    cp.wait()
                    pltpu.sync_copy(mb, slab.at[dx], add=True)
                    return 0

                lax.fori_loop(0, NB, chunk, 0)
                plsc.subcore_barrier()
                pltpu.sync_copy(slab.at[pl.ds(myz0, SLABR // NS)],
                                out_hbm.at[pl.ds(lo + myz0, SLABR // NS)])
                plsc.subcore_barrier()

    return k(msgw, dst)[:N]


def _segsum(h128, src, dst, w):
    msg = _gather_edges(h128, src)
    msgw = _tc_scale(msg, w)
    return _sc_scatter(msgw, dst)


def kernel(mask, edge_index, edge_weight, sw, bert_feat, embed_table,
           W1, b1, W2, b2, W3, b3, Wb, bb, gamma, beta):
    src = edge_index[0]
    dst = edge_index[1]

    h0 = _gather_nodes(embed_table, mask)
    agg1 = _segsum(h0, src, dst, edge_weight)
    u1a, u1b = _tc_layer1(h0, agg1, W1, b1)
    agg2a = _segsum(u1a, src, dst, edge_weight)
    agg2b = _segsum(u1b, src, dst, edge_weight)
    m = _tc_layer23(u1a, u1b, agg2a, agg2b, W2, b2, W3)
    agg3 = _segsum(m, src, dst, edge_weight)
    psum = _tc_layer3_pool(m, agg3, b3)
    psum = psum.reshape(NGRAPH, -1, D0)
    return _tc_mix(psum, bert_feat, Wb, bb, gamma, beta)


# speedup vs baseline: 1.0139x; 1.0139x over previous
import functools

import jax
import jax.numpy as jnp
from jax import lax
from jax.experimental import pallas as pl
from jax.experimental.pallas import tpu as pltpu
from jax.experimental.pallas import tpu_sc as plsc

N_PER_GRAPH = 16384
NGRAPH = 4
N = NGRAPH * N_PER_GRAPH
E = 524288
D0 = 128
D1 = 256

NC = 2
NS = 16
NW = NC * NS
F = 128
FC = 128
FC2 = 256
SLABR = 6656
NSLAB = (N + SLABR - 1) // SLABR
ZR = 32


_ERF_P = 0.3275911
_ERF_A1 = 0.254829592
_ERF_A2 = -0.284496736
_ERF_A3 = 1.421413741
_ERF_A4 = -1.453152027
_ERF_A5 = 1.061405429


def _gelu(x):
    z = jnp.abs(x) * 0.7071067811865476
    t = 1.0 / (1.0 + _ERF_P * z)
    poly = t * (_ERF_A1 + t * (_ERF_A2 + t * (_ERF_A3 + t * (_ERF_A4 + t * _ERF_A5))))
    erf = jnp.sign(x) * (1.0 - poly * jnp.exp(-z * z))
    return 0.5 * x * (1.0 + erf)


def _tc_scale(msg, w):
    BLK = 8192

    def body(m_ref, w_ref, o_ref):
        o_ref[...] = m_ref[...] * w_ref[...]

    return pl.pallas_call(
        body,
        grid=(E // BLK,),
        in_specs=[
            pl.BlockSpec((BLK, D0), lambda i: (i, 0)),
            pl.BlockSpec((BLK, 1), lambda i: (i, 0)),
        ],
        out_specs=pl.BlockSpec((BLK, D0), lambda i: (i, 0)),
        out_shape=jax.ShapeDtypeStruct((E, D0), jnp.float32),
    )(msg, w.reshape(E, 1))


def _tc_layer1(h, agg, W, b):
    BLK = 2048

    def body(h_ref, a_ref, w_ref, b_ref, oa_ref, ob_ref):
        x = h_ref[...] + a_ref[...]
        y = _gelu(jnp.dot(x, w_ref[...], preferred_element_type=jnp.float32)
                  + b_ref[...])
        oa_ref[...] = y[:, :D0]
        ob_ref[...] = y[:, D0:]

    return pl.pallas_call(
        body,
        grid=(N // BLK,),
        in_specs=[
            pl.BlockSpec((BLK, D0), lambda i: (i, 0)),
            pl.BlockSpec((BLK, D0), lambda i: (i, 0)),
            pl.BlockSpec((D0, D1), lambda i: (0, 0)),
            pl.BlockSpec((1, D1), lambda i: (0, 0)),
        ],
        out_specs=[
            pl.BlockSpec((BLK, D0), lambda i: (i, 0)),
            pl.BlockSpec((BLK, D0), lambda i: (i, 0)),
        ],
        out_shape=[
            jax.ShapeDtypeStruct((N, D0), jnp.float32),
            jax.ShapeDtypeStruct((N, D0), jnp.float32),
        ],
    )(h, agg, W, b.reshape(1, D1))


def _tc_layer23(u1a, u1b, agg2a, agg2b, W2, b2, W3):
    BLK = 2048

    def body(ha_ref, hb_ref, aa_ref, ab_ref, w2a_ref, w2b_ref, b2_ref,
             w3_ref, o_ref):
        xa = ha_ref[...] + aa_ref[...]
        xb = hb_ref[...] + ab_ref[...]
        y = jnp.dot(xa, w2a_ref[...], preferred_element_type=jnp.float32)
        y = y + jnp.dot(xb, w2b_ref[...], preferred_element_type=jnp.float32)
        y = _gelu(y + b2_ref[...])
        o_ref[...] = jnp.dot(y, w3_ref[...], preferred_element_type=jnp.float32)

    return pl.pallas_call(
        body,
        grid=(N // BLK,),
        in_specs=[
            pl.BlockSpec((BLK, D0), lambda i: (i, 0)),
            pl.BlockSpec((BLK, D0), lambda i: (i, 0)),
            pl.BlockSpec((BLK, D0), lambda i: (i, 0)),
            pl.BlockSpec((BLK, D0), lambda i: (i, 0)),
            pl.BlockSpec((D0, D1), lambda i: (0, 0)),
            pl.BlockSpec((D0, D1), lambda i: (0, 0)),
            pl.BlockSpec((1, D1), lambda i: (0, 0)),
            pl.BlockSpec((D1, D0), lambda i: (0, 0)),
        ],
        out_specs=pl.BlockSpec((BLK, D0), lambda i: (i, 0)),
        out_shape=jax.ShapeDtypeStruct((N, D0), jnp.float32),
    )(u1a, u1b, agg2a, agg2b, W2[:D0], W2[D0:], b2.reshape(1, D1), W3)


def _tc_layer3_pool(m, agg, b3):
    BLK = 2048
    NBLK = N // BLK

    def body(m_ref, a_ref, b_ref, o_ref):
        y = _gelu(m_ref[...] + a_ref[...] + b_ref[...])
        o_ref[...] = jnp.sum(y, axis=0).reshape(1, 1, D0)

    return pl.pallas_call(
        body,
        grid=(NBLK,),
        in_specs=[
            pl.BlockSpec((BLK, D0), lambda i: (i, 0)),
            pl.BlockSpec((BLK, D0), lambda i: (i, 0)),
            pl.BlockSpec((1, D0), lambda i: (0, 0)),
        ],
        out_specs=pl.BlockSpec((1, 1, D0), lambda i: (i, 0, 0)),
        out_shape=jax.ShapeDtypeStruct((NBLK, 1, D0), jnp.float32),
    )(m, agg, b3.reshape(1, D0))


def _tc_mix(psum, bert_feat, Wb, bb, gamma, beta):
    def body(ps_ref, bf_ref, wb_ref, bb_ref, g_ref, be_ref, o_ref):
        pool = jnp.sum(ps_ref[...], axis=1) * (1.0 / N_PER_GRAPH)
        enc = jnp.dot(bf_ref[...], wb_ref[...],
                      preferred_element_type=jnp.float32) + bb_ref[...]
        mu = jnp.mean(enc, axis=-1, keepdims=True)
        var = jnp.mean((enc - mu) ** 2, axis=-1, keepdims=True)
        enc = (enc - mu) / jnp.sqrt(var + 1e-5) * g_ref[...] + be_ref[...]
        o_ref[...] = (pool + enc) * 0.5

    nblk = psum.shape[1]
    return pl.pallas_call(
        body,
        in_specs=[
            pl.BlockSpec((NGRAPH, nblk, D0), lambda: (0, 0, 0)),
            pl.BlockSpec((NGRAPH, 1536), lambda: (0, 0)),
            pl.BlockSpec((1536, D0), lambda: (0, 0)),
            pl.BlockSpec((1, D0), lambda: (0, 0)),
            pl.BlockSpec((1, D0), lambda: (0, 0)),
            pl.BlockSpec((1, D0), lambda: (0, 0)),
        ],
        out_specs=pl.BlockSpec((NGRAPH, D0), lambda: (0, 0)),
        out_shape=jax.ShapeDtypeStruct((NGRAPH, D0), jnp.float32),
    )(psum, bert_feat, Wb, bb.reshape(1, D0), gamma.reshape(1, D0),
      beta.reshape(1, D0))


def _make_gather(NROWS):
    rows_per_w = NROWS // NW
    n_ch = rows_per_w // F
    mesh = plsc.VectorSubcoreMesh(core_axis_name="c", subcore_axis_name="s")

    @functools.partial(
        pl.kernel,
        mesh=mesh,
        out_type=jax.ShapeDtypeStruct((NROWS, D0), jnp.float32),
        scratch_types=[
            pltpu.VMEM((F,), jnp.int32),
            pltpu.VMEM((F, D0), jnp.float32),
            pltpu.VMEM((F,), jnp.int32),
            pltpu.VMEM((F, D0), jnp.float32),
            pltpu.SemaphoreType.DMA,
            pltpu.SemaphoreType.DMA,
        ],
    )
    def k(table_hbm, idx_hbm, out_hbm, ia, ra, ib, rb, sa, sb):
        wid = lax.axis_index("s") * NC + lax.axis_index("c")
        base = wid * rows_per_w

        def body(i, _):
            offa = base + (2 * i) * F
            offb = offa + F
            pltpu.sync_copy(idx_hbm.at[pl.ds(offa, F)], ia)
            cpa = pltpu.async_copy(table_hbm.at[ia], ra, sa)
            pltpu.sync_copy(idx_hbm.at[pl.ds(offb, F)], ib)
            cpb = pltpu.async_copy(table_hbm.at[ib], rb, sb)
            cpa.wait()
            pltpu.sync_copy(ra, out_hbm.at[pl.ds(offa, F)])
            cpb.wait()
            pltpu.sync_copy(rb, out_hbm.at[pl.ds(offb, F)])
            return 0

        lax.fori_loop(0, n_ch // 2, body, 0)

    return k


_gather_nodes = _make_gather(N)
_gather_edges = _make_gather(E)


def _sc_scatter(msgw, dst):
    ER = E // NS
    NP = ER // (2 * FC2)
    mesh = plsc.VectorSubcoreMesh(core_axis_name="c", subcore_axis_name="s")

    @functools.partial(
        pl.kernel,
        mesh=mesh,
        out_type=jax.ShapeDtypeStruct((NSLAB * SLABR, D0), jnp.float32),
        scratch_types=[
            pltpu.VMEM((FC2,), jnp.int32),
            pltpu.VMEM((FC,), jnp.int32),
            pltpu.VMEM((FC,), jnp.int32),
            pltpu.VMEM((FC,), jnp.int32),
            pltpu.VMEM((FC,), jnp.int32),
            pltpu.VMEM((FC2, D0), jnp.float32),
            pltpu.VMEM((FC2, D0), jnp.float32),
            pltpu.VMEM((ZR, D0), jnp.float32),
            pltpu.VMEM_SHARED((SLABR + 8, D0), jnp.float32),
            pltpu.SemaphoreType.DMA,
            pltpu.SemaphoreType.DMA,
            pltpu.SemaphoreType.DMA,
            pltpu.SemaphoreType.DMA,
        ],
    )
    def k(msg_hbm, dst_hbm, out_hbm, dstblk, dxa0, dxa1, dxb0, dxb1,
          mba, mbb, zbuf, slab, sema, semb, semsa, semsb):
        c = lax.axis_index("c")
        s = lax.axis_index("s")
        zero16 = jnp.zeros((16,), jnp.float32)
        ebase = s * ER

        def zrow(r, _):
            for ccs in range(D0 // 16):
                zbuf[r, pl.ds(ccs * 16, 16)] = zero16
            return 0

        lax.fori_loop(0, ZR, zrow, 0)
        myz0 = s * (SLABR // NS)

        def prep(lo, base_q, dx0, dx1):
            for q in range(FC2 // 16):
                dl = dstblk[pl.ds(q * 16, 16)] - lo
                ok = (dl >= 0) & (dl < SLABR)
                v = jnp.where(ok, dl, SLABR)
                if q < FC // 16:
                    dx0[pl.ds(q * 16, 16)] = v
                else:
                    dx1[pl.ds((q - FC // 16) * 16, 16)] = v
            del base_q

        for g in range(NSLAB):
            lo = g * SLABR

            @pl.when(c == (g % NC))
            def _():
                for zi in range(SLABR // NS // ZR):
                    pltpu.sync_copy(zbuf, slab.at[pl.ds(myz0 + zi * ZR, ZR)])
                plsc.subcore_barrier()
                pltpu.async_copy(msg_hbm.at[pl.ds(ebase, FC2)], mba, sema)

                def chunk(p, _):
                    g0 = ebase + (2 * p) * FC2
                    g1 = g0 + FC2
                    cpb = pltpu.async_copy(msg_hbm.at[pl.ds(g1, FC2)], mbb,
                                           semb)
                    pltpu.sync_copy(dst_hbm.at[pl.ds(g0, FC2)], dstblk)
                    prep(lo, 0, dxa0, dxa1)
                    pltpu.make_async_copy(msg_hbm.at[pl.ds(g0, FC2)], mba,
                                          sema).wait()
                    sa0 = pltpu.async_copy(mba.at[pl.ds(0, FC)],
                                           slab.at[dxa0], semsa, add=True)
                    sa1 = pltpu.async_copy(mba.at[pl.ds(FC, FC)],
                                           slab.at[dxa1], semsa, add=True)
                    pltpu.sync_copy(dst_hbm.at[pl.ds(g1, FC2)], dstblk)
                    prep(lo, 0, dxb0, dxb1)
                    cpb.wait()
                    sb0 = pltpu.async_copy(mbb.at[pl.ds(0, FC)],
                                           slab.at[dxb0], semsb, add=True)
                    sb1 = pltpu.async_copy(mbb.at[pl.ds(FC, FC)],
                                           slab.at[dxb1], semsb, add=True)
                    sa0.wait()
                    sa1.wait()
                    gn = jnp.minimum(g0 + 2 * FC2, ebase + ER - FC2)
                    pltpu.async_copy(msg_hbm.at[pl.ds(gn, FC2)], mba, sema)
                    sb0.wait()
                    sb1.wait()
                    return 0

                lax.fori_loop(0, NP, chunk, 0)
                pltpu.make_async_copy(msg_hbm.at[pl.ds(ebase, FC2)], mba,
                                      sema).wait()
                plsc.subcore_barrier()
                pltpu.sync_copy(slab.at[pl.ds(myz0, SLABR // NS)],
                                out_hbm.at[pl.ds(lo + myz0, SLABR // NS)])
                plsc.subcore_barrier()

    return k(msgw, dst)[:N]


def _segsum(h128, src, dst, w):
    msg = _gather_edges(h128, src)
    msgw = _tc_scale(msg, w)
    return _sc_scatter(msgw, dst)


def kernel(mask, edge_index, edge_weight, sw, bert_feat, embed_table,
           W1, b1, W2, b2, W3, b3, Wb, bb, gamma, beta):
    src = edge_index[0]
    dst = edge_index[1]

    h0 = _gather_nodes(embed_table, mask)
    agg1 = _segsum(h0, src, dst, edge_weight)
    u1a, u1b = _tc_layer1(h0, agg1, W1, b1)
    agg2a = _segsum(u1a, src, dst, edge_weight)
    agg2b = _segsum(u1b, src, dst, edge_weight)
    m = _tc_layer23(u1a, u1b, agg2a, agg2b, W2, b2, W3)
    agg3 = _segsum(m, src, dst, edge_weight)
    psum = _tc_layer3_pool(m, agg3, b3)
    psum = psum.reshape(NGRAPH, -1, D0)
    return _tc_mix(psum, bert_feat, Wb, bb, gamma, beta)
